# baseline (speedup 1.0000x reference)
import functools

import jax
import jax.numpy as jnp
from jax import lax
from jax.experimental import pallas as pl
from jax.experimental.pallas import tpu as pltpu
from jax.experimental.pallas import tpu_sc as plsc

N_ROWS = 16384
D = 64
L = 16
NUM_CORES = 2
NUM_SUBCORES = 16
NW = NUM_CORES * NUM_SUBCORES
THRESHOLD = 0.01

C_TC = 8192
S_SC = N_ROWS - C_TC
SC_ROWS_PER_W = S_SC // NW
TC_B = 512


def _row_topk_softmax(x):
  s0 = lax.sort(x[0])
  s1 = lax.sort(x[1])
  s2 = lax.sort(x[2])
  s3 = lax.sort(x[3])
  h1 = jnp.maximum(s0, lax.rev(s1, (0,)))
  h2 = jnp.maximum(s2, lax.rev(s3, (0,)))
  h = jnp.maximum(lax.sort(h1), lax.rev(lax.sort(h2), (0,)))
  hs = lax.sort(h)
  m = hs[jnp.full((L,), 15, jnp.int32)]
  t8 = hs[jnp.full((L,), 8, jnp.int32)]
  e = [jnp.exp(v - m) for v in x]
  s = jnp.sum((e[0] + e[1]) + (e[2] + e[3]))
  inv = jnp.full((L,), 1.0, jnp.float32) / jnp.broadcast_to(s, (L,))
  p = [v * inv for v in e]
  thr = jnp.float32(THRESHOLD)
  return [
      jnp.where((v >= t8) & (q >= thr), q, jnp.float32(0.0))
      for v, q in zip(x, p)
  ]


def _sc_body(x_hbm, out_hbm, in_v, out_v):
  wid = lax.axis_index("s") * NUM_CORES + lax.axis_index("c")
  base = wid * SC_ROWS_PER_W
  pltpu.sync_copy(x_hbm.at[pl.ds(base, SC_ROWS_PER_W)], in_v)

  def row_step(r):
    x = [in_v[r, pl.ds(16 * j, L)] for j in range(4)]
    o = _row_topk_softmax(x)
    for j in range(4):
      out_v[r, pl.ds(16 * j, L)] = o[j]

  plsc.parallel_loop(0, SC_ROWS_PER_W, 1, unroll=2)(row_step)

  pltpu.sync_copy(out_v, out_hbm.at[pl.ds(base, SC_ROWS_PER_W)])


def _sc_half(x):
  mesh = plsc.VectorSubcoreMesh(core_axis_name="c", subcore_axis_name="s")
  f = pl.kernel(
      _sc_body,
      out_type=jax.ShapeDtypeStruct((S_SC, D), jnp.float32),
      mesh=mesh,
      scratch_types=[
          pltpu.VMEM((SC_ROWS_PER_W, D), jnp.float32),
          pltpu.VMEM((SC_ROWS_PER_W, D), jnp.float32),
      ],
      compiler_params=pltpu.CompilerParams(
          needs_layout_passes=False, use_tc_tiling_on_sc=True),
  )
  return f(x)


def _tc_body(x_ref, o_ref):
  x = x_ref[...]
  t = jnp.full((1, TC_B), jnp.inf, jnp.float32)
  cnt = jnp.zeros((1, TC_B), jnp.float32)
  for _ in range(8):
    nm = jnp.max(jnp.where(x < t, x, -jnp.inf), axis=0, keepdims=True)
    c = jnp.sum(jnp.where(x == nm, 1.0, 0.0), axis=0, keepdims=True)
    upd = cnt < 8.0
    t = jnp.where(upd, nm, t)
    cnt = cnt + jnp.where(upd, c, 0.0)
  m = jnp.max(x, axis=0, keepdims=True)
  e = jnp.exp(x - m)
  s = jnp.sum(e, axis=0, keepdims=True)
  p = e / s
  o_ref[...] = jnp.where((x >= t) & (p >= jnp.float32(THRESHOLD)), p,
                         jnp.float32(0.0))


def _tc_half(x_t):
  return pl.pallas_call(
      _tc_body,
      grid=(C_TC // TC_B,),
      in_specs=[pl.BlockSpec((D, TC_B), lambda i: (0, i))],
      out_specs=pl.BlockSpec((D, TC_B), lambda i: (0, i)),
      out_shape=jax.ShapeDtypeStruct((D, C_TC), jnp.float32),
  )(x_t)


@jax.jit
def kernel(inputs):
  x_t = inputs.T
  out_tc_t = _tc_half(x_t)
  out_sc = _sc_half(lax.slice(inputs, (C_TC, 0), (N_ROWS, D)))
  cat = jnp.concatenate([out_tc_t, out_sc.T], axis=1)
  return cat.T


# speedup vs baseline: 1.1393x; 1.1393x over previous
import functools

import jax
import jax.numpy as jnp
from jax import lax
from jax.experimental import pallas as pl
from jax.experimental.pallas import tpu as pltpu
from jax.experimental.pallas import tpu_sc as plsc

N_ROWS = 16384
D = 64
L = 16
NUM_CORES = 2
NUM_SUBCORES = 16
NW = NUM_CORES * NUM_SUBCORES
THRESHOLD = 0.01

C_TC = 0
S_SC = N_ROWS - C_TC
SC_ROWS_PER_W = S_SC // NW
TC_B = 512


def _row_topk_softmax(x):
  s0 = lax.sort(x[0])
  s1 = lax.sort(x[1])
  s2 = lax.sort(x[2])
  s3 = lax.sort(x[3])
  h1 = jnp.maximum(s0, lax.rev(s1, (0,)))
  h2 = jnp.maximum(s2, lax.rev(s3, (0,)))
  h = jnp.maximum(lax.sort(h1), lax.rev(lax.sort(h2), (0,)))
  hs = lax.sort(h)
  m = hs[jnp.full((L,), 15, jnp.int32)]
  t8 = hs[jnp.full((L,), 8, jnp.int32)]
  e = [jnp.exp(v - m) for v in x]
  s = jnp.sum((e[0] + e[1]) + (e[2] + e[3]))
  inv = jnp.full((L,), 1.0, jnp.float32) / jnp.broadcast_to(s, (L,))
  p = [v * inv for v in e]
  thr = jnp.float32(THRESHOLD)
  return [
      jnp.where((v >= t8) & (q >= thr), q, jnp.float32(0.0))
      for v, q in zip(x, p)
  ]


def _sc_body(x_hbm, out_hbm, in_v, out_v):
  wid = lax.axis_index("s") * NUM_CORES + lax.axis_index("c")
  base = wid * SC_ROWS_PER_W
  pltpu.sync_copy(x_hbm.at[pl.ds(base, SC_ROWS_PER_W)], in_v)

  def row_step(r):
    x = [in_v[r, pl.ds(16 * j, L)] for j in range(4)]
    o = _row_topk_softmax(x)
    for j in range(4):
      out_v[r, pl.ds(16 * j, L)] = o[j]

  plsc.parallel_loop(0, SC_ROWS_PER_W, 1, unroll=2)(row_step)

  pltpu.sync_copy(out_v, out_hbm.at[pl.ds(base, SC_ROWS_PER_W)])


def _sc_half(x):
  mesh = plsc.VectorSubcoreMesh(core_axis_name="c", subcore_axis_name="s")
  f = pl.kernel(
      _sc_body,
      out_type=jax.ShapeDtypeStruct((S_SC, D), jnp.float32),
      mesh=mesh,
      scratch_types=[
          pltpu.VMEM((SC_ROWS_PER_W, D), jnp.float32),
          pltpu.VMEM((SC_ROWS_PER_W, D), jnp.float32),
      ],
      compiler_params=pltpu.CompilerParams(
          needs_layout_passes=False, use_tc_tiling_on_sc=True),
  )
  return f(x)


def _tc_body(x_ref, o_ref):
  x = x_ref[...]
  t = jnp.full((1, TC_B), jnp.inf, jnp.float32)
  cnt = jnp.zeros((1, TC_B), jnp.float32)
  for _ in range(8):
    nm = jnp.max(jnp.where(x < t, x, -jnp.inf), axis=0, keepdims=True)
    c = jnp.sum(jnp.where(x == nm, 1.0, 0.0), axis=0, keepdims=True)
    upd = cnt < 8.0
    t = jnp.where(upd, nm, t)
    cnt = cnt + jnp.where(upd, c, 0.0)
  m = jnp.max(x, axis=0, keepdims=True)
  e = jnp.exp(x - m)
  s = jnp.sum(e, axis=0, keepdims=True)
  p = e / s
  o_ref[...] = jnp.where((x >= t) & (p >= jnp.float32(THRESHOLD)), p,
                         jnp.float32(0.0))


def _tc_half(x_t):
  return pl.pallas_call(
      _tc_body,
      grid=(C_TC // TC_B,),
      in_specs=[pl.BlockSpec((D, TC_B), lambda i: (0, i))],
      out_specs=pl.BlockSpec((D, TC_B), lambda i: (0, i)),
      out_shape=jax.ShapeDtypeStruct((D, C_TC), jnp.float32),
  )(x_t)


@jax.jit
def kernel(inputs):
  return _sc_half(inputs)
